# baseline (speedup 1.0000x reference)
import functools

import jax
import jax.numpy as jnp
from jax import lax
from jax.experimental import pallas as pl
from jax.experimental.pallas import tpu as pltpu
from jax.experimental.pallas import tpu_sc as plsc

N = 10000
D = 128
H = 64
E = 320000
NC, NS = 2, 16
NW = NC * NS
NP = 10240
RPT = NP // NS
ZCH = 128
CH = 128
NCHUNK = 80
EPW = NCHUNK * CH
EPAD = NW * EPW
NB = 2

_f32 = jnp.float32


def _sc_bag3(tables, src2d, dst2d, zrows, zcnt, with_counts):
    mesh = plsc.VectorSubcoreMesh(core_axis_name="c", subcore_axis_name="s",
                                  num_cores=NC, num_subcores=NS)
    out_type = [jax.ShapeDtypeStruct((3, NC, NP, H), _f32)]
    if with_counts:
        out_type.append(jax.ShapeDtypeStruct((3, NC, NS, NP), _f32))
    scratch = {
        "acc": pltpu.VMEM_SHARED((NP, H), _f32),
        "src_v": pltpu.VMEM((NCHUNK, CH), jnp.int32),
        "dst_v": pltpu.VMEM((NCHUNK, CH), jnp.int32),
        "bufs": pltpu.VMEM((NB, CH, H), _f32),
        "zrows_v": pltpu.VMEM((ZCH, H), _f32),
    }
    for b in range(NB):
        scratch[f"gsem{b}"] = pltpu.SemaphoreType.DMA
        scratch[f"ssem{b}"] = pltpu.SemaphoreType.DMA
    if with_counts:
        scratch.update({
            "hist_v": pltpu.VMEM((NP,), _f32),
        })

    def body(t0, t1, t2, s0, s1, s2, d0, d1, d2, zr_h,
             zc_h, *outs_and_scratch):
        if with_counts:
            out_acc, out_cnt = outs_and_scratch[:2]
            sc = dict(zip(scratch.keys(), outs_and_scratch[2:]))
        else:
            out_acc = outs_and_scratch[0]
            sc = dict(zip(scratch.keys(), outs_and_scratch[1:]))
        c = lax.axis_index("c")
        s = lax.axis_index("s")
        pltpu.sync_copy(zr_h, sc["zrows_v"])

        for t, (tab, sv, dv) in enumerate(
                zip((t0, t1, t2), (s0, s1, s2), (d0, d1, d2))):
            for k in range(RPT // ZCH):
                pltpu.sync_copy(sc["zrows_v"],
                                sc["acc"].at[pl.ds(s * RPT + k * ZCH, ZCH)])
            if with_counts:
                pltpu.sync_copy(zc_h, sc["hist_v"])
            plsc.subcore_barrier()

            pltpu.sync_copy(sv.at[c, s], sc["src_v"])
            pltpu.sync_copy(dv.at[c, s], sc["dst_v"])

            def gstart(j, u):
                pltpu.async_copy(tab.at[sc["src_v"].at[j]],
                                 sc["bufs"].at[u], sc[f"gsem{u}"])

            def gwait(j, u):
                pltpu.make_async_copy(tab.at[sc["src_v"].at[j]],
                                      sc["bufs"].at[u],
                                      sc[f"gsem{u}"]).wait()

            def scat(j, u):
                pltpu.sync_copy(sc["bufs"].at[u],
                                sc["acc"].at[sc["dst_v"].at[j]],
                                add=True)

            gstart(0, 0)

            def pair(g, carry):
                j0 = g * 2
                gstart(j0 + 1, 1)
                gwait(j0, 0)
                scat(j0, 0)

                @pl.when(g + 1 < NCHUNK // 2)
                def _():
                    gstart(j0 + 2, 0)

                gwait(j0 + 1, 1)
                scat(j0 + 1, 1)
                return carry
            lax.fori_loop(0, NCHUNK // 2, pair, 0)

            if with_counts:
                ones = jnp.ones((16,), _f32)

                def cbody(i, carry):
                    d = sc["dst_v"][i // 8, pl.ds((i % 8) * 16, 16)]
                    plsc.addupdate_scatter(sc["hist_v"], [d], ones)
                    return carry
                lax.fori_loop(0, EPW // 16, cbody, 0)

            plsc.subcore_barrier()

            pltpu.sync_copy(sc["acc"].at[pl.ds(s * RPT, RPT)],
                            out_acc.at[t, c, pl.ds(s * RPT, RPT)])
            if with_counts:
                pltpu.sync_copy(sc["hist_v"], out_cnt.at[t, c, s])
            plsc.subcore_barrier()

    kfn = pl.kernel(body, out_type=out_type, mesh=mesh,
                    scratch_types=list(scratch.values()),
                    compiler_params=pltpu.CompilerParams(
                        needs_layout_passes=False,
                        use_tc_tiling_on_sc=False))
    return kfn(*tables, *src2d, *dst2d, zrows, zcnt)


_BR = 512
_GRID = 20


def _full(shape):
    return pl.BlockSpec(shape, lambda i: (0,) * len(shape))


def _rows(w):
    return pl.BlockSpec((_BR, w), lambda i: (i, 0))


def _dot(a, b):
    return jax.lax.dot(a, b, preferred_element_type=_f32)


def _tc1_body(xu, xi, wrl, wal, wvl, wrr, war, wvr, br, ba, bv,
              tr, ta, tv, d_i, d_u):
    tr[...] = _dot(xu[...], wrl[...])
    ta[...] = _dot(xi[...], wal[...])
    tv[...] = _dot(xi[...], wvl[...])
    d_i[...] = _dot(xi[...], wrr[...] + war[...]) + br[...] + ba[...]
    d_u[...] = _dot(xu[...], wvr[...]) + bv[...]


def _tc1(xu, xi, wrl, wal, wvl, wrr, war, wvr, br, ba, bv):
    o = jax.ShapeDtypeStruct((N, H), _f32)
    return pl.pallas_call(
        _tc1_body,
        grid=(_GRID,),
        in_specs=[_rows(D), _rows(D)] + [_full((D, H))] * 6 + [_full((1, H))] * 3,
        out_specs=[_rows(H)] * 5,
        out_shape=[o] * 5,
    )(xu, xi, wrl, wal, wvl, wrr, war, wvr,
      br.reshape(1, H), ba.reshape(1, H), bv.reshape(1, H))


def _means(acc, cnt):
    tot = jnp.sum(cnt, axis=(1, 2))
    inv = (1.0 / jnp.maximum(tot, 1.0))[:, :, None]
    return [(acc[t, 0] + acc[t, 1]) * inv[t] for t in range(3)]


def _acc_spec():
    return pl.BlockSpec((3, NC, _BR, H), lambda i: (0, 0, i, 0))


def _cnt_spec():
    return pl.BlockSpec((3, NC, NS, _BR), lambda i: (0, 0, 0, i))


def _tc2_body(acc, cnt, d_i, d_u, wrl, wal, wvl, wrr, war, wvr, br, ba, bv,
              t2r, t2a, t2v, d2i, d2u):
    mr, ma, mv = _means(acc[...], cnt[...])
    item1 = jax.nn.relu(mr + ma + d_i[...])
    user1 = jax.nn.relu(mv + d_u[...])
    t2r[...] = _dot(user1, wrl[...])
    t2a[...] = _dot(item1, wal[...])
    t2v[...] = _dot(item1, wvl[...])
    d2i[...] = _dot(item1, wrr[...] + war[...]) + br[...] + ba[...]
    d2u[...] = _dot(user1, wvr[...]) + bv[...]


def _tc2(acc, cnt, d_i, d_u, wrl, wal, wvl, wrr, war, wvr, br, ba, bv):
    o = jax.ShapeDtypeStruct((N, H), _f32)
    return pl.pallas_call(
        _tc2_body,
        grid=(_GRID,),
        in_specs=[_acc_spec(), _cnt_spec(), _rows(H), _rows(H)]
        + [_full((H, H))] * 6 + [_full((1, H))] * 3,
        out_specs=[_rows(H)] * 5,
        out_shape=[o] * 5,
    )(acc, cnt, d_i, d_u, wrl, wal, wvl, wrr, war, wvr,
      br.reshape(1, H), ba.reshape(1, H), bv.reshape(1, H))


def _tc3_body(acc, cnt, d2i, d2u, user2, item2):
    mr, ma, mv = _means(acc[...], cnt[...])
    item2[...] = mr + ma + d2i[...]
    user2[...] = mv + d2u[...]


def _tc3(acc, cnt, d2i, d2u):
    o = jax.ShapeDtypeStruct((N, H), _f32)
    return pl.pallas_call(
        _tc3_body,
        grid=(_GRID,),
        in_specs=[_acc_spec(), _cnt_spec(), _rows(H), _rows(H)],
        out_specs=[_rows(H)] * 2,
        out_shape=[o] * 2,
    )(acc, cnt, d2i, d2u)


def kernel(x_user, x_item, edge_reviews, edge_rev_reviews, edge_also_bought,
           W1r_l, b1r, W1r_r, W1v_l, b1v, W1v_r, W1a_l, b1a, W1a_r,
           W2r_l, b2r, W2r_r, W2v_l, b2v, W2v_r, W2a_l, b2a, W2a_r):
    src2d, dst2d = [], []
    dfill = N + (jnp.arange(EPAD - E, dtype=jnp.int32) % (NP - N))
    for e in (edge_reviews, edge_also_bought, edge_rev_reviews):
        e = e.astype(jnp.int32)
        srcp = jnp.pad(e[0], (0, EPAD - E))
        dstp = jnp.concatenate([e[1], dfill])
        src2d.append(srcp.reshape(NC, NS, NCHUNK, CH))
        dst2d.append(dstp.reshape(NC, NS, NCHUNK, CH))
    zrows = jnp.zeros((ZCH, H), _f32)
    zcnt = jnp.zeros((NP,), _f32)

    tr, ta, tv, d1i, d1u = _tc1(x_user, x_item, W1r_l, W1a_l, W1v_l,
                                W1r_r, W1a_r, W1v_r, b1r, b1a, b1v)
    acc1, cnt4 = _sc_bag3((tr, ta, tv), src2d, dst2d, zrows,
                          zcnt, with_counts=True)
    t2r, t2a, t2v, d2i, d2u = _tc2(acc1, cnt4, d1i, d1u, W2r_l, W2a_l, W2v_l,
                                   W2r_r, W2a_r, W2v_r, b2r, b2a, b2v)
    (acc2,) = _sc_bag3((t2r, t2a, t2v), src2d, dst2d, zrows,
                       zcnt, with_counts=False)
    user2, item2 = _tc3(acc2, cnt4, d2i, d2u)
    return (user2, item2)


# speedup vs baseline: 2.4592x; 2.4592x over previous
import functools

import jax
import jax.numpy as jnp
from jax import lax
from jax.experimental import pallas as pl
from jax.experimental.pallas import tpu as pltpu
from jax.experimental.pallas import tpu_sc as plsc

N = 10000
D = 128
H = 64
E = 320000
NC, NS = 2, 16
NW = NC * NS
NP = 10240
RPT = NP // NS
ZCH = 128
CH = 125
NCHUNK = 80
EPW = NCHUNK * CH
NB = 2

_f32 = jnp.float32


def _sc_bag3(tables, src2d, dst2d, dstf, zrows, zcnt, with_counts):
    mesh = plsc.VectorSubcoreMesh(core_axis_name="c", subcore_axis_name="s",
                                  num_cores=NC, num_subcores=NS)
    out_type = [jax.ShapeDtypeStruct((3, NC, NP, H), _f32)]
    if with_counts:
        out_type.append(jax.ShapeDtypeStruct((3, NC, NS, NP), _f32))
    scratch = {
        "acc": pltpu.VMEM_SHARED((NP, H), _f32),
        "src_v": pltpu.VMEM((NCHUNK, CH), jnp.int32),
        "dst_v": pltpu.VMEM((NCHUNK, CH), jnp.int32),
        "rows_a": pltpu.VMEM((CH, H), _f32),
        "rows_b": pltpu.VMEM((CH, H), _f32),
        "zrows_v": pltpu.VMEM((ZCH, H), _f32),
        "sem_a": pltpu.SemaphoreType.DMA,
        "sem_b": pltpu.SemaphoreType.DMA,
    }
    if with_counts:
        scratch.update({
            "dstf_v": pltpu.VMEM((EPW,), jnp.int32),
            "hist_v": pltpu.VMEM((NP,), _f32),
        })

    def body(t0, t1, t2, s0, s1, s2, d0, d1, d2, f0, f1, f2, zr_h,
             zc_h, *outs_and_scratch):
        if with_counts:
            out_acc, out_cnt = outs_and_scratch[:2]
            sc = dict(zip(scratch.keys(), outs_and_scratch[2:]))
        else:
            out_acc = outs_and_scratch[0]
            sc = dict(zip(scratch.keys(), outs_and_scratch[1:]))
        c = lax.axis_index("c")
        s = lax.axis_index("s")
        pltpu.sync_copy(zr_h, sc["zrows_v"])

        for t, (tab, sv, dv, fv) in enumerate(
                zip((t0, t1, t2), (s0, s1, s2), (d0, d1, d2), (f0, f1, f2))):
            for k in range(RPT // ZCH):
                pltpu.sync_copy(sc["zrows_v"],
                                sc["acc"].at[pl.ds(s * RPT + k * ZCH, ZCH)])
            if with_counts:
                pltpu.sync_copy(zc_h, sc["hist_v"])
            plsc.subcore_barrier()

            pltpu.sync_copy(sv.at[c, s], sc["src_v"])
            pltpu.sync_copy(dv.at[c, s], sc["dst_v"])
            if with_counts:
                pltpu.sync_copy(fv.at[c, s], sc["dstf_v"])

            def gstart(j, buf, sem):
                pltpu.async_copy(tab.at[sc["src_v"].at[j]], buf, sem)

            def gwait(j, buf, sem):
                pltpu.make_async_copy(tab.at[sc["src_v"].at[j]], buf,
                                      sem).wait()

            def scat(j, buf):
                pltpu.sync_copy(buf, sc["acc"].at[sc["dst_v"].at[j]],
                                add=True)

            gstart(0, sc["rows_a"], sc["sem_a"])

            def pair(g, carry):
                j0 = g * 2
                gstart(j0 + 1, sc["rows_b"], sc["sem_b"])
                gwait(j0, sc["rows_a"], sc["sem_a"])
                scat(j0, sc["rows_a"])

                @pl.when(g + 1 < NCHUNK // 2)
                def _():
                    gstart(j0 + 2, sc["rows_a"], sc["sem_a"])

                gwait(j0 + 1, sc["rows_b"], sc["sem_b"])
                scat(j0 + 1, sc["rows_b"])
                return carry
            lax.fori_loop(0, NCHUNK // 2, pair, 0)

            if with_counts:
                ones = jnp.ones((16,), _f32)

                def cbody(i, carry):
                    d = sc["dstf_v"][pl.ds(i * 16, 16)]
                    plsc.addupdate_scatter(sc["hist_v"], [d], ones)
                    return carry
                lax.fori_loop(0, EPW // 16, cbody, 0)

            plsc.subcore_barrier()

            pltpu.sync_copy(sc["acc"].at[pl.ds(s * RPT, RPT)],
                            out_acc.at[t, c, pl.ds(s * RPT, RPT)])
            if with_counts:
                pltpu.sync_copy(sc["hist_v"], out_cnt.at[t, c, s])
            plsc.subcore_barrier()

    kfn = pl.kernel(body, out_type=out_type, mesh=mesh,
                    scratch_types=list(scratch.values()),
                    compiler_params=pltpu.CompilerParams(
                        needs_layout_passes=False,
                        use_tc_tiling_on_sc=False))
    return kfn(*tables, *src2d, *dst2d, *dstf, zrows, zcnt)


_BR = 512
_GRID = 20


def _full(shape):
    return pl.BlockSpec(shape, lambda i: (0,) * len(shape))


def _rows(w):
    return pl.BlockSpec((_BR, w), lambda i: (i, 0))


def _dot(a, b):
    return jax.lax.dot(a, b, preferred_element_type=_f32)


def _tc1_body(xu, xi, wrl, wal, wvl, wrr, war, wvr, br, ba, bv,
              tr, ta, tv, d_i, d_u):
    tr[...] = _dot(xu[...], wrl[...])
    ta[...] = _dot(xi[...], wal[...])
    tv[...] = _dot(xi[...], wvl[...])
    d_i[...] = _dot(xi[...], wrr[...] + war[...]) + br[...] + ba[...]
    d_u[...] = _dot(xu[...], wvr[...]) + bv[...]


def _tc1(xu, xi, wrl, wal, wvl, wrr, war, wvr, br, ba, bv):
    o = jax.ShapeDtypeStruct((N, H), _f32)
    return pl.pallas_call(
        _tc1_body,
        grid=(_GRID,),
        in_specs=[_rows(D), _rows(D)] + [_full((D, H))] * 6 + [_full((1, H))] * 3,
        out_specs=[_rows(H)] * 5,
        out_shape=[o] * 5,
    )(xu, xi, wrl, wal, wvl, wrr, war, wvr,
      br.reshape(1, H), ba.reshape(1, H), bv.reshape(1, H))


def _means(acc, cnt):
    tot = jnp.sum(cnt, axis=(1, 2))
    inv = (1.0 / jnp.maximum(tot, 1.0))[:, :, None]
    return [(acc[t, 0] + acc[t, 1]) * inv[t] for t in range(3)]


def _acc_spec():
    return pl.BlockSpec((3, NC, _BR, H), lambda i: (0, 0, i, 0))


def _cnt_spec():
    return pl.BlockSpec((3, NC, NS, _BR), lambda i: (0, 0, 0, i))


def _tc2_body(acc, cnt, d_i, d_u, wrl, wal, wvl, wrr, war, wvr, br, ba, bv,
              t2r, t2a, t2v, d2i, d2u):
    mr, ma, mv = _means(acc[...], cnt[...])
    item1 = jax.nn.relu(mr + ma + d_i[...])
    user1 = jax.nn.relu(mv + d_u[...])
    t2r[...] = _dot(user1, wrl[...])
    t2a[...] = _dot(item1, wal[...])
    t2v[...] = _dot(item1, wvl[...])
    d2i[...] = _dot(item1, wrr[...] + war[...]) + br[...] + ba[...]
    d2u[...] = _dot(user1, wvr[...]) + bv[...]


def _tc2(acc, cnt, d_i, d_u, wrl, wal, wvl, wrr, war, wvr, br, ba, bv):
    o = jax.ShapeDtypeStruct((N, H), _f32)
    return pl.pallas_call(
        _tc2_body,
        grid=(_GRID,),
        in_specs=[_acc_spec(), _cnt_spec(), _rows(H), _rows(H)]
        + [_full((H, H))] * 6 + [_full((1, H))] * 3,
        out_specs=[_rows(H)] * 5,
        out_shape=[o] * 5,
    )(acc, cnt, d_i, d_u, wrl, wal, wvl, wrr, war, wvr,
      br.reshape(1, H), ba.reshape(1, H), bv.reshape(1, H))


def _tc3_body(acc, cnt, d2i, d2u, user2, item2):
    mr, ma, mv = _means(acc[...], cnt[...])
    item2[...] = mr + ma + d2i[...]
    user2[...] = mv + d2u[...]


def _tc3(acc, cnt, d2i, d2u):
    o = jax.ShapeDtypeStruct((N, H), _f32)
    return pl.pallas_call(
        _tc3_body,
        grid=(_GRID,),
        in_specs=[_acc_spec(), _cnt_spec(), _rows(H), _rows(H)],
        out_specs=[_rows(H)] * 2,
        out_shape=[o] * 2,
    )(acc, cnt, d2i, d2u)


def kernel(x_user, x_item, edge_reviews, edge_rev_reviews, edge_also_bought,
           W1r_l, b1r, W1r_r, W1v_l, b1v, W1v_r, W1a_l, b1a, W1a_r,
           W2r_l, b2r, W2r_r, W2v_l, b2v, W2v_r, W2a_l, b2a, W2a_r):
    src2d, dst2d, dstf = [], [], []
    for e in (edge_reviews, edge_also_bought, edge_rev_reviews):
        e = e.astype(jnp.int32)
        src2d.append(e[0].reshape(NC, NS, NCHUNK, CH))
        dst2d.append(e[1].reshape(NC, NS, NCHUNK, CH))
        dstf.append(e[1].reshape(NC, NS, EPW))
    zrows = jnp.zeros((ZCH, H), _f32)
    zcnt = jnp.zeros((NP,), _f32)

    tr, ta, tv, d1i, d1u = _tc1(x_user, x_item, W1r_l, W1a_l, W1v_l,
                                W1r_r, W1a_r, W1v_r, b1r, b1a, b1v)
    acc1, cnt4 = _sc_bag3((tr, ta, tv), src2d, dst2d, dstf, zrows,
                          zcnt, with_counts=True)
    t2r, t2a, t2v, d2i, d2u = _tc2(acc1, cnt4, d1i, d1u, W2r_l, W2a_l, W2v_l,
                                   W2r_r, W2a_r, W2v_r, b2r, b2a, b2v)
    (acc2,) = _sc_bag3((t2r, t2a, t2v), src2d, dst2d, dstf, zrows,
                       zcnt, with_counts=False)
    user2, item2 = _tc3(acc2, cnt4, d2i, d2u)
    return (user2, item2)
